# baseline (speedup 1.0000x reference)
import functools

import jax
import jax.numpy as jnp
from jax import lax
from jax.experimental import pallas as pl
from jax.experimental.pallas import tpu as pltpu
from jax.experimental.pallas import tpu_sc as plsc

_NC, _NS = 2, 16
_NW = _NC * _NS
_BATCH, _FEAT = 4096, 128
_ROWS_PER_W = _BATCH // _NW


@functools.partial(
    pl.kernel,
    out_type=jax.ShapeDtypeStruct((_BATCH, _FEAT), jnp.float32),
    mesh=plsc.VectorSubcoreMesh(core_axis_name="c", subcore_axis_name="s"),
    scratch_types=[pltpu.SemaphoreType.DMA],
)
def _sc_copy(x_hbm, out_hbm, sem):
    wid = lax.axis_index("s") * _NC + lax.axis_index("c")
    base = wid * _ROWS_PER_W
    pltpu.async_copy(
        x_hbm.at[pl.ds(base, _ROWS_PER_W)],
        out_hbm.at[pl.ds(base, _ROWS_PER_W)],
        sem,
    ).wait()


def kernel(x, queue):
    del queue
    return _sc_copy(x)


# speedup vs baseline: 31.4965x; 31.4965x over previous
import jax
import jax.numpy as jnp
from jax.experimental import pallas as pl
from jax.experimental.pallas import tpu as pltpu

_N_CHUNKS = 8
_ROWS = 4096 // _N_CHUNKS


def _copy_kernel(x_ref, o_ref, scratch, in_sems, out_sems):
    for i in range(_N_CHUNKS):
        pltpu.make_async_copy(
            x_ref.at[pl.ds(i * _ROWS, _ROWS)], scratch.at[i], in_sems.at[i]
        ).start()
    for i in range(_N_CHUNKS):
        pltpu.make_async_copy(
            x_ref.at[pl.ds(i * _ROWS, _ROWS)], scratch.at[i], in_sems.at[i]
        ).wait()
        pltpu.make_async_copy(
            scratch.at[i], o_ref.at[pl.ds(i * _ROWS, _ROWS)], out_sems.at[i]
        ).start()
    for i in range(_N_CHUNKS):
        pltpu.make_async_copy(
            scratch.at[i], o_ref.at[pl.ds(i * _ROWS, _ROWS)], out_sems.at[i]
        ).wait()


def kernel(x, queue):
    del queue
    return pl.pallas_call(
        _copy_kernel,
        in_specs=[pl.BlockSpec(memory_space=pl.ANY)],
        out_specs=pl.BlockSpec(memory_space=pl.ANY),
        out_shape=jax.ShapeDtypeStruct(x.shape, x.dtype),
        scratch_shapes=[
            pltpu.VMEM((_N_CHUNKS, _ROWS, x.shape[1]), x.dtype),
            pltpu.SemaphoreType.DMA((_N_CHUNKS,)),
            pltpu.SemaphoreType.DMA((_N_CHUNKS,)),
        ],
    )(x)
